# Initial kernel scaffold; baseline (speedup 1.0000x reference)
import jax
import jax.numpy as jnp
from jax.experimental import pallas as pl


def kernel(fen, move, rank_emb, file_emb, fen_emb, move_emb, abs_emb):
    raise NotImplementedError("write your pallas kernel here")


# speedup vs baseline: 1.3554x; 1.3554x over previous
import functools

import jax
import jax.numpy as jnp
from jax import lax
from jax.experimental import pallas as pl
from jax.experimental.pallas import tpu as pltpu
from jax.experimental.pallas import tpu_sc as plsc

D = 1024
B = 4096
NROW = 71
NIDX = 72
NC, NS = 2, 16
NW = NC * NS
BPW = B // NW
RPW = BPW * NROW
IPW = BPW * NIDX
LANES = 16


def _sc_body(G, idxs, boardc, out, idx_v, base_v, buf_v, sem):
    wid = lax.axis_index("s") * NC + lax.axis_index("c")
    row0 = wid * RPW
    pltpu.sync_copy(idxs.at[pl.ds(wid * IPW, IPW)], idx_v)
    pltpu.sync_copy(boardc, base_v)

    def do_chunk(ilocal, local, j0, jb, add_base):
        pltpu.async_copy(
            G.at[idx_v.at[pl.ds(ilocal + j0, jb)]],
            buf_v.at[pl.ds(0, jb)],
            sem,
        ).wait()
        if add_base:
            def row_body(r, _):
                def vec_body(v, _):
                    o = v * LANES
                    buf_v[r, pl.ds(o, LANES)] = (
                        buf_v[r, pl.ds(o, LANES)] + base_v[j0 + r, pl.ds(o, LANES)]
                    )
                    return 0
                lax.fori_loop(0, D // LANES, vec_body, 0, unroll=4)
                return 0
            lax.fori_loop(0, jb, row_body, 0)
        pltpu.sync_copy(
            buf_v.at[pl.ds(0, jb)],
            out.at[pl.ds(row0 + local + j0, jb)],
        )

    def batch_body(b, _):
        ilocal = b * NIDX
        local = b * NROW
        do_chunk(ilocal, local, 0, 32, True)
        do_chunk(ilocal, local, 32, 32, True)
        do_chunk(ilocal, local, 64, 7, False)
        return 0

    lax.fori_loop(0, BPW, batch_body, 0)


@functools.partial(
    pl.kernel,
    out_type=jax.ShapeDtypeStruct((B * NROW, D), jnp.float32),
    mesh=plsc.VectorSubcoreMesh(
        core_axis_name="c", subcore_axis_name="s", num_cores=NC, num_subcores=NS
    ),
    compiler_params=pltpu.CompilerParams(use_tc_tiling_on_sc=False),
    scratch_types=[
        pltpu.VMEM((IPW,), jnp.int32),
        pltpu.VMEM((64, D), jnp.float32),
        pltpu.VMEM((32, D), jnp.float32),
        pltpu.SemaphoreType.DMA,
    ],
)
def _sc_embed(G, idxs, boardc, out, idx_v, base_v, buf_v, sem):
    _sc_body(G, idxs, boardc, out, idx_v, base_v, buf_v, sem)


def kernel(fen, move, rank_emb, file_emb, fen_emb, move_emb, abs_emb):
    pos = (rank_emb + file_emb).reshape(64, D)
    pair = 0.5 * (fen_emb[:, None, :] + fen_emb[None, :, :]).reshape(17 * 17, D)
    flag_tab = (fen_emb[None, :, :] + abs_emb[64:69][:, None, :]).reshape(5 * 17, D)
    mv_tab = (
        0.58 * (pos[None, :, :] + move_emb[:, None, :])
        + abs_emb[69:71][:, None, :]
    ).reshape(2 * 64, D)
    G = jnp.concatenate([pair, flag_tab, mv_tab], axis=0)
    boardc = 0.5 * pos + abs_emb[:64]

    idx_board = fen[:, :64] * 17 + fen[:, 64:128]
    idx_flag = 289 + jnp.arange(5, dtype=jnp.int32) * 17 + fen[:, 128:133]
    idx_mv = 374 + jnp.arange(2, dtype=jnp.int32) * 64 + move
    idx_pad = jnp.zeros((B, NIDX - NROW), dtype=jnp.int32)
    idx = jnp.concatenate([idx_board, idx_flag, idx_mv, idx_pad], axis=1).reshape(-1)

    out = _sc_embed(G, idx, boardc)
    return out.reshape(B, NROW, D)


# speedup vs baseline: 2.5448x; 1.8775x over previous
import functools

import jax
import jax.numpy as jnp
from jax import lax
from jax.experimental import pallas as pl
from jax.experimental.pallas import tpu as pltpu
from jax.experimental.pallas import tpu_sc as plsc

D = 1024
B = 4096
NROW = 71
NC, NS = 2, 16
NW = NC * NS
RPW = (B * NROW) // NW

NBLK = 293
GROWS = NBLK * 64

CH = 16
NB = 4
LAG = 2
NCHUNK = RPW // CH
NGRP = NCHUNK // NB


def _build_body(pair_ref, add_ref, out_ref):
    out_ref[...] = pair_ref[0] + add_ref[...]


def _build_table(pair_ext, addend):
    return pl.pallas_call(
        _build_body,
        grid=(NBLK,),
        in_specs=[
            pl.BlockSpec((1, 1, D), lambda k: (k, 0, 0)),
            pl.BlockSpec((64, D), lambda k: (jnp.where(k < 289, 0, k - 288), 0)),
        ],
        out_specs=pl.BlockSpec((64, D), lambda k: (k, 0)),
        out_shape=jax.ShapeDtypeStruct((GROWS, D), jnp.float32),
    )(pair_ext.reshape(NBLK, 1, D), addend)


def _sc_body(G2, idxs, out, idx_v, bufs, gsem, wsem):
    wid = lax.axis_index("s") * NC + lax.axis_index("c")
    row0 = wid * RPW
    pltpu.sync_copy(idxs.at[pl.ds(row0, RPW)], idx_v)

    def gather(i, s):
        off = pl.multiple_of(i * CH, CH)
        return pltpu.make_async_copy(
            G2.at[idx_v.at[pl.ds(off, CH)]], bufs.at[s], gsem.at[s])

    def write(i, s):
        off = pl.multiple_of(row0 + i * CH, CH)
        return pltpu.make_async_copy(
            bufs.at[s], out.at[pl.ds(off, CH)], wsem.at[s])

    for s in range(LAG):
        gather(s, s).start()

    def group(g, _):
        i0 = g * NB
        for s in range(NB):
            i = i0 + s
            s2 = (s + LAG) % NB

            @pl.when(i + LAG < NCHUNK)
            def _():
                @pl.when(i + LAG >= NB)
                def _():
                    write(i + LAG - NB, s2).wait()
                gather(i + LAG, s2).start()

            gather(i, s).wait()
            write(i, s).start()
        return 0

    lax.fori_loop(0, NGRP, group, 0)
    for s in range(NB):
        write(NCHUNK - NB + s, s).wait()


@functools.partial(
    pl.kernel,
    out_type=jax.ShapeDtypeStruct((B * NROW, D), jnp.float32),
    mesh=plsc.VectorSubcoreMesh(
        core_axis_name="c", subcore_axis_name="s", num_cores=NC, num_subcores=NS
    ),
    compiler_params=pltpu.CompilerParams(use_tc_tiling_on_sc=False),
    scratch_types=[
        pltpu.VMEM((RPW,), jnp.int32),
        pltpu.VMEM((NB, CH, D), jnp.float32),
        pltpu.SemaphoreType.DMA((NB,)),
        pltpu.SemaphoreType.DMA((NB,)),
    ],
)
def _sc_embed(G2, idxs, out, idx_v, bufs, gsem, wsem):
    _sc_body(G2, idxs, out, idx_v, bufs, gsem, wsem)


def kernel(fen, move, rank_emb, file_emb, fen_emb, move_emb, abs_emb):
    pos = (rank_emb + file_emb).reshape(64, D)
    pair = 0.5 * (fen_emb[:, None, :] + fen_emb[None, :, :]).reshape(289, D)
    boardc = 0.5 * pos + abs_emb[:64]
    flag_tab = (fen_emb[None, :, :] + abs_emb[64:69][:, None, :]).reshape(85, D)
    mv_tab = (
        0.58 * (pos[None, :, :] + move_emb[:, None, :])
        + abs_emb[69:71][:, None, :]
    ).reshape(128, D)
    pair_ext = jnp.concatenate([pair, jnp.zeros((4, D), jnp.float32)])
    addend = jnp.concatenate(
        [boardc, flag_tab, mv_tab, jnp.zeros((43, D), jnp.float32)])
    G2 = _build_table(pair_ext, addend)

    iota64 = jnp.arange(64, dtype=jnp.int32)
    idx_board = (fen[:, :64] * 17 + fen[:, 64:128]) * 64 + iota64
    idx_flag = 18496 + jnp.arange(5, dtype=jnp.int32) * 17 + fen[:, 128:133]
    idx_mv = 18581 + jnp.arange(2, dtype=jnp.int32) * 64 + move
    idx = jnp.concatenate([idx_board, idx_flag, idx_mv], axis=1).reshape(-1)

    out = _sc_embed(G2, idx)
    return out.reshape(B, NROW, D)


# speedup vs baseline: 2.8945x; 1.1374x over previous
import functools

import jax
import jax.numpy as jnp
from jax import lax
from jax.experimental import pallas as pl
from jax.experimental.pallas import tpu as pltpu
from jax.experimental.pallas import tpu_sc as plsc

D = 1024
B = 4096
NROW = 71
NC, NS = 2, 16
NW = NC * NS
RPW = (B * NROW) // NW

NBLK = 293
GROWS = NBLK * 64

CH = 16
NB = 4
LAG = 2
NCHUNK = RPW // CH
NGRP = NCHUNK // NB


def _build_body(pair_ref, add_ref, out_ref):
    out_ref[...] = pair_ref[0] + add_ref[...]


def _build_table(pair_ext, addend):
    return pl.pallas_call(
        _build_body,
        grid=(NBLK,),
        in_specs=[
            pl.BlockSpec((1, 1, D), lambda k: (k, 0, 0)),
            pl.BlockSpec((64, D), lambda k: (jnp.where(k < 289, 0, k - 288), 0)),
        ],
        out_specs=pl.BlockSpec((64, D), lambda k: (k, 0)),
        out_shape=jax.ShapeDtypeStruct((GROWS, D), jnp.float32),
    )(pair_ext.reshape(NBLK, 1, D), addend)


def _sc_body(G2, idxs, out, idx_v, bufs, gsem, wsem):
    wid = lax.axis_index("s") * NC + lax.axis_index("c")
    row0 = wid * RPW
    pltpu.sync_copy(idxs.at[pl.ds(row0, RPW)], idx_v)

    def gather(i, s):
        off = pl.multiple_of(i * CH, CH)
        return pltpu.make_async_copy(
            G2.at[idx_v.at[pl.ds(off, CH)]], bufs.at[s], gsem.at[s])

    def write(i, s):
        off = pl.multiple_of(row0 + i * CH, CH)
        return pltpu.make_async_copy(
            bufs.at[s], out.at[pl.ds(off, CH)], wsem.at[s])

    for s in range(LAG):
        gather(s, s).start()

    def group(g, _):
        i0 = g * NB
        for s in range(NB):
            i = i0 + s
            s2 = (s + LAG) % NB

            @pl.when(i + LAG < NCHUNK)
            def _():
                @pl.when(i + LAG >= NB)
                def _():
                    write(i + LAG - NB, s2).wait()
                gather(i + LAG, s2).start()

            gather(i, s).wait()
            write(i, s).start()
        return 0

    lax.fori_loop(0, NGRP, group, 0)
    for s in range(NB):
        write(NCHUNK - NB + s, s).wait()


@functools.partial(
    pl.kernel,
    out_type=jax.ShapeDtypeStruct((B * NROW, D), jnp.float32),
    mesh=plsc.VectorSubcoreMesh(
        core_axis_name="c", subcore_axis_name="s", num_cores=NC, num_subcores=NS
    ),
    compiler_params=pltpu.CompilerParams(use_tc_tiling_on_sc=True),
    scratch_types=[
        pltpu.VMEM((RPW,), jnp.int32),
        pltpu.VMEM((NB, CH, D), jnp.float32),
        pltpu.SemaphoreType.DMA((NB,)),
        pltpu.SemaphoreType.DMA((NB,)),
    ],
)
def _sc_embed(G2, idxs, out, idx_v, bufs, gsem, wsem):
    _sc_body(G2, idxs, out, idx_v, bufs, gsem, wsem)


def kernel(fen, move, rank_emb, file_emb, fen_emb, move_emb, abs_emb):
    pos = (rank_emb + file_emb).reshape(64, D)
    pair = 0.5 * (fen_emb[:, None, :] + fen_emb[None, :, :]).reshape(289, D)
    boardc = 0.5 * pos + abs_emb[:64]
    flag_tab = (fen_emb[None, :, :] + abs_emb[64:69][:, None, :]).reshape(85, D)
    mv_tab = (
        0.58 * (pos[None, :, :] + move_emb[:, None, :])
        + abs_emb[69:71][:, None, :]
    ).reshape(128, D)
    pair_ext = jnp.concatenate([pair, jnp.zeros((4, D), jnp.float32)])
    addend = jnp.concatenate(
        [boardc, flag_tab, mv_tab, jnp.zeros((43, D), jnp.float32)])
    G2 = _build_table(pair_ext, addend)

    iota64 = jnp.arange(64, dtype=jnp.int32)
    idx_board = (fen[:, :64] * 17 + fen[:, 64:128]) * 64 + iota64
    idx_flag = 18496 + jnp.arange(5, dtype=jnp.int32) * 17 + fen[:, 128:133]
    idx_mv = 18581 + jnp.arange(2, dtype=jnp.int32) * 64 + move
    idx = jnp.concatenate([idx_board, idx_flag, idx_mv], axis=1).reshape(-1)

    out = _sc_embed(G2, idx)
    return out.reshape(B, NROW, D)


# speedup vs baseline: 3.6145x; 1.2487x over previous
import functools

import jax
import jax.numpy as jnp
from jax import lax
from jax.experimental import pallas as pl
from jax.experimental.pallas import tpu as pltpu
from jax.experimental.pallas import tpu_sc as plsc

D = 1024
B = 4096
NROW = 71
NIDX = 72
NC, NS = 2, 16
NW = NC * NS
BPW = B // NW
IPW = BPW * NIDX

NBLK = 293
GROWS = NBLK * 64

CHUNKS = ((0, 32, 32), (32, 32, 32), (64, 8, 7))


def _build_body(pair_ref, add_ref, out_ref):
    out_ref[...] = pair_ref[0] + add_ref[...]


def _build_table(pair_ext, addend):
    return pl.pallas_call(
        _build_body,
        grid=(NBLK,),
        in_specs=[
            pl.BlockSpec((1, 1, D), lambda k: (k, 0, 0)),
            pl.BlockSpec((64, D), lambda k: (jnp.where(k < 289, 0, k - 288), 0)),
        ],
        out_specs=pl.BlockSpec((64, D), lambda k: (k, 0)),
        out_shape=jax.ShapeDtypeStruct((GROWS, D), jnp.float32),
    )(pair_ext.reshape(NBLK, 1, D), addend)


def _sc_body(G2, idxs, out, idx_v, bufs, buf2, gsem, wsem):
    wid = lax.axis_index("s") * NC + lax.axis_index("c")
    pltpu.sync_copy(idxs.at[pl.ds(wid * IPW, IPW)], idx_v)
    bb0 = wid * BPW

    def gather(b, k):
        j0, gb, _ = CHUNKS[k]
        off = pl.multiple_of(b * NIDX + j0, 8)
        dst = bufs.at[k] if gb == 32 else buf2
        return pltpu.make_async_copy(
            G2.at[idx_v.at[pl.ds(off, gb)]], dst, gsem.at[k])

    def write(b, k):
        j0, gb, wb = CHUNKS[k]
        src = bufs.at[k] if gb == 32 else buf2.at[pl.ds(0, wb)]
        return pltpu.make_async_copy(
            src, out.at[bb0 + b, pl.ds(j0, wb)], wsem.at[k])

    for k in range(3):
        gather(0, k).start()

    def elem(b, _):
        for k in range(3):
            gather(b, k).wait()
            write(b, k).start()
            write(b, k).wait()

            @pl.when(b < BPW - 1)
            def _():
                gather(b + 1, k).start()
        return 0

    lax.fori_loop(0, BPW, elem, 0)


@functools.partial(
    pl.kernel,
    out_type=jax.ShapeDtypeStruct((B, NROW, D), jnp.float32),
    mesh=plsc.VectorSubcoreMesh(
        core_axis_name="c", subcore_axis_name="s", num_cores=NC, num_subcores=NS
    ),
    compiler_params=pltpu.CompilerParams(use_tc_tiling_on_sc=True),
    scratch_types=[
        pltpu.VMEM((IPW,), jnp.int32),
        pltpu.VMEM((3, 32, D), jnp.float32),
        pltpu.VMEM((8, D), jnp.float32),
        pltpu.SemaphoreType.DMA((3,)),
        pltpu.SemaphoreType.DMA((3,)),
    ],
)
def _sc_embed(G2, idxs, out, idx_v, bufs, buf2, gsem, wsem):
    _sc_body(G2, idxs, out, idx_v, bufs, buf2, gsem, wsem)


def kernel(fen, move, rank_emb, file_emb, fen_emb, move_emb, abs_emb):
    pos = (rank_emb + file_emb).reshape(64, D)
    pair = 0.5 * (fen_emb[:, None, :] + fen_emb[None, :, :]).reshape(289, D)
    boardc = 0.5 * pos + abs_emb[:64]
    flag_tab = (fen_emb[None, :, :] + abs_emb[64:69][:, None, :]).reshape(85, D)
    mv_tab = (
        0.58 * (pos[None, :, :] + move_emb[:, None, :])
        + abs_emb[69:71][:, None, :]
    ).reshape(128, D)
    pair_ext = jnp.concatenate([pair, jnp.zeros((4, D), jnp.float32)])
    addend = jnp.concatenate(
        [boardc, flag_tab, mv_tab, jnp.zeros((43, D), jnp.float32)])
    G2 = _build_table(pair_ext, addend)

    iota64 = jnp.arange(64, dtype=jnp.int32)
    idx_board = (fen[:, :64] * 17 + fen[:, 64:128]) * 64 + iota64
    idx_flag = 18496 + jnp.arange(5, dtype=jnp.int32) * 17 + fen[:, 128:133]
    idx_mv = 18581 + jnp.arange(2, dtype=jnp.int32) * 64 + move
    idx_pad = jnp.zeros((B, NIDX - NROW), dtype=jnp.int32)
    idx = jnp.concatenate(
        [idx_board, idx_flag, idx_mv, idx_pad], axis=1).reshape(-1)

    return _sc_embed(G2, idx)


# speedup vs baseline: 7.5348x; 2.0846x over previous
import functools

import jax
import jax.numpy as jnp
from jax import lax
from jax.experimental import pallas as pl
from jax.experimental.pallas import tpu as pltpu
from jax.experimental.pallas import tpu_sc as plsc

D = 1024
B = 4096
NROW = 71
NC, NS = 2, 16
NW = NC * NS
RPW = (B * NROW) // NW

NBLK = 293
GROWS = NBLK * 64

CH = 16
NB = 4
LAG = 2
NCHUNK = RPW // CH
NGRP = NCHUNK // NB


def _build_body(pair_ref, add_ref, out_ref):
    out_ref[...] = pair_ref[0] + add_ref[...]


def _build_table(pair_ext, addend):
    return pl.pallas_call(
        _build_body,
        grid=(NBLK,),
        in_specs=[
            pl.BlockSpec((1, 1, D), lambda k: (k, 0, 0)),
            pl.BlockSpec((64, D), lambda k: (jnp.where(k < 289, 0, k - 288), 0)),
        ],
        out_specs=pl.BlockSpec((64, D), lambda k: (k, 0)),
        out_shape=jax.ShapeDtypeStruct((GROWS, D), jnp.float32),
    )(pair_ext.reshape(NBLK, 1, D), addend)


def _sc_body(G2, idxs, out, idx_v, bufs, gsem, wsem):
    wid = lax.axis_index("s") * NC + lax.axis_index("c")
    row0 = wid * RPW
    pltpu.sync_copy(idxs.at[pl.ds(row0, RPW)], idx_v)

    def gather(i, s):
        off = pl.multiple_of(i * CH, CH)
        return pltpu.make_async_copy(
            G2.at[idx_v.at[pl.ds(off, CH)]], bufs.at[s], gsem.at[s])

    def write(i, s):
        off = pl.multiple_of(row0 + i * CH, CH)
        return pltpu.make_async_copy(
            bufs.at[s], out.at[pl.ds(off, CH)], wsem.at[s])

    for s in range(LAG):
        gather(s, s).start()

    def group(g, _):
        i0 = g * NB
        for s in range(NB):
            i = i0 + s
            s2 = (s + LAG) % NB

            @pl.when(i + LAG < NCHUNK)
            def _():
                @pl.when(i + LAG >= NB)
                def _():
                    write(i + LAG - NB, s2).wait()
                gather(i + LAG, s2).start()

            gather(i, s).wait()
            write(i, s).start()
        return 0

    lax.fori_loop(0, NGRP, group, 0)
    for s in range(NB):
        write(NCHUNK - NB + s, s).wait()


@functools.partial(
    pl.kernel,
    out_type=jax.ShapeDtypeStruct((NROW * B, D), jnp.float32),
    mesh=plsc.VectorSubcoreMesh(
        core_axis_name="c", subcore_axis_name="s", num_cores=NC, num_subcores=NS
    ),
    compiler_params=pltpu.CompilerParams(use_tc_tiling_on_sc=True),
    scratch_types=[
        pltpu.VMEM((RPW,), jnp.int32),
        pltpu.VMEM((NB, CH, D), jnp.float32),
        pltpu.SemaphoreType.DMA((NB,)),
        pltpu.SemaphoreType.DMA((NB,)),
    ],
)
def _sc_embed(G2, idxs, out, idx_v, bufs, gsem, wsem):
    _sc_body(G2, idxs, out, idx_v, bufs, gsem, wsem)


def kernel(fen, move, rank_emb, file_emb, fen_emb, move_emb, abs_emb):
    pos = (rank_emb + file_emb).reshape(64, D)
    pair = 0.5 * (fen_emb[:, None, :] + fen_emb[None, :, :]).reshape(289, D)
    boardc = 0.5 * pos + abs_emb[:64]
    flag_tab = (fen_emb[None, :, :] + abs_emb[64:69][:, None, :]).reshape(85, D)
    mv_tab = (
        0.58 * (pos[None, :, :] + move_emb[:, None, :])
        + abs_emb[69:71][:, None, :]
    ).reshape(128, D)
    pair_ext = jnp.concatenate([pair, jnp.zeros((4, D), jnp.float32)])
    addend = jnp.concatenate(
        [boardc, flag_tab, mv_tab, jnp.zeros((43, D), jnp.float32)])
    G2 = _build_table(pair_ext, addend)

    iota64 = jnp.arange(64, dtype=jnp.int32)
    idx_board = (fen[:, :64] * 17 + fen[:, 64:128]) * 64 + iota64
    idx_flag = 18496 + jnp.arange(5, dtype=jnp.int32) * 17 + fen[:, 128:133]
    idx_mv = 18581 + jnp.arange(2, dtype=jnp.int32) * 64 + move
    idx = jnp.concatenate(
        [idx_board, idx_flag, idx_mv], axis=1).T.reshape(-1)

    out = _sc_embed(G2, idx)
    return out.reshape(NROW, B, D).transpose(1, 0, 2)


# speedup vs baseline: 8.2253x; 1.0917x over previous
import functools

import jax
import jax.numpy as jnp
from jax import lax
from jax.experimental import pallas as pl
from jax.experimental.pallas import tpu as pltpu
from jax.experimental.pallas import tpu_sc as plsc

D = 1024
B = 4096
NROW = 71
NC, NS = 2, 16
NW = NC * NS
RPW = (B * NROW) // NW

NBLK = 38
GROWS = NBLK * 8 * 64
SMALL0 = 37 * 8 * 64

CH = 16
NB = 4
LAG = 2
NCHUNK = RPW // CH
NGRP = NCHUNK // NB


def _build_body(pair_ref, add_ref, out_ref):
    out_ref[...] = pair_ref[...] + add_ref[...]


def _build_table(pair_ext, addend3):
    out3 = pl.pallas_call(
        _build_body,
        grid=(NBLK,),
        in_specs=[
            pl.BlockSpec((8, 1, D), lambda k: (k, 0, 0)),
            pl.BlockSpec((8, 64, D), lambda k: (jnp.where(k < 37, 0, 1), 0, 0)),
        ],
        out_specs=pl.BlockSpec((8, 64, D), lambda k: (k, 0, 0)),
        out_shape=jax.ShapeDtypeStruct((NBLK * 8, 64, D), jnp.float32),
    )(pair_ext.reshape(NBLK * 8, 1, D), addend3)
    return out3.reshape(GROWS, D)


def _sc_body(G2, idxs, out, idx_v, bufs, gsem, wsem):
    wid = lax.axis_index("s") * NC + lax.axis_index("c")
    row0 = wid * RPW
    pltpu.sync_copy(idxs.at[pl.ds(row0, RPW)], idx_v)

    def gather(i, s):
        off = pl.multiple_of(i * CH, CH)
        return pltpu.make_async_copy(
            G2.at[idx_v.at[pl.ds(off, CH)]], bufs.at[s], gsem.at[s])

    def write(i, s):
        off = pl.multiple_of(row0 + i * CH, CH)
        return pltpu.make_async_copy(
            bufs.at[s], out.at[pl.ds(off, CH)], wsem.at[s])

    for s in range(LAG):
        gather(s, s).start()

    def group(g, _):
        i0 = g * NB
        for s in range(NB):
            i = i0 + s
            s2 = (s + LAG) % NB

            @pl.when(i + LAG < NCHUNK)
            def _():
                @pl.when(i + LAG >= NB)
                def _():
                    write(i + LAG - NB, s2).wait()
                gather(i + LAG, s2).start()

            gather(i, s).wait()
            write(i, s).start()
        return 0

    lax.fori_loop(0, NGRP, group, 0)
    for s in range(NB):
        write(NCHUNK - NB + s, s).wait()


@functools.partial(
    pl.kernel,
    out_type=jax.ShapeDtypeStruct((NROW * B, D), jnp.float32),
    mesh=plsc.VectorSubcoreMesh(
        core_axis_name="c", subcore_axis_name="s", num_cores=NC, num_subcores=NS
    ),
    compiler_params=pltpu.CompilerParams(use_tc_tiling_on_sc=True),
    scratch_types=[
        pltpu.VMEM((RPW,), jnp.int32),
        pltpu.VMEM((NB, CH, D), jnp.float32),
        pltpu.SemaphoreType.DMA((NB,)),
        pltpu.SemaphoreType.DMA((NB,)),
    ],
)
def _sc_embed(G2, idxs, out, idx_v, bufs, gsem, wsem):
    _sc_body(G2, idxs, out, idx_v, bufs, gsem, wsem)


def kernel(fen, move, rank_emb, file_emb, fen_emb, move_emb, abs_emb):
    pos = (rank_emb + file_emb).reshape(64, D)
    pair = 0.5 * (fen_emb[:, None, :] + fen_emb[None, :, :]).reshape(289, D)
    boardc = 0.5 * pos + abs_emb[:64]
    flag_tab = (fen_emb[None, :, :] + abs_emb[64:69][:, None, :]).reshape(85, D)
    mv_tab = (
        0.58 * (pos[None, :, :] + move_emb[:, None, :])
        + abs_emb[69:71][:, None, :]
    ).reshape(128, D)
    pair_ext = jnp.concatenate(
        [pair, jnp.zeros((NBLK * 8 - 289, D), jnp.float32)])
    small_pad = jnp.concatenate(
        [flag_tab, mv_tab, jnp.zeros((512 - 85 - 128, D), jnp.float32)])
    addend3 = jnp.stack(
        [jnp.broadcast_to(boardc, (8, 64, D)), small_pad.reshape(8, 64, D)]
    ).reshape(16, 64, D)
    G2 = _build_table(pair_ext, addend3)

    iota64 = jnp.arange(64, dtype=jnp.int32)
    idx_board = (fen[:, :64] * 17 + fen[:, 64:128]) * 64 + iota64
    idx_flag = SMALL0 + jnp.arange(5, dtype=jnp.int32) * 17 + fen[:, 128:133]
    idx_mv = SMALL0 + 85 + jnp.arange(2, dtype=jnp.int32) * 64 + move
    idx = jnp.concatenate(
        [idx_board, idx_flag, idx_mv], axis=1).T.reshape(-1)

    out = _sc_embed(G2, idx)
    return out.reshape(NROW, B, D).transpose(1, 0, 2)
